# baseline (speedup 1.0000x reference)
import jax
import jax.numpy as jnp
from jax import lax
from jax.experimental import pallas as pl
from jax.experimental.pallas import tpu as pltpu
from jax.experimental.pallas import tpu_sc as plsc

N = 10000
E = 320000
D = 128

NC = 2
NS = 16
NW = NC * NS

CH = 80
CHUNKS = 125
H = CHUNKS
EPT = CH * CHUNKS
E_PAD = EPT * NW
DPT = 632
N_ACC = DPT * NS

_mesh = plsc.VectorSubcoreMesh(core_axis_name="c", subcore_axis_name="s")


def _deg_body(dst_hbm, out_hbm, didx, ones_v, zbuf, sem, acc):
    c = lax.axis_index("c")
    s = lax.axis_index("s")
    t = c * NS + s
    pltpu.sync_copy(dst_hbm.at[t], didx)
    for k in range(CH // 16):
        ones_v[pl.ds(k * 16, 16)] = jnp.ones((16,), jnp.float32)
    for k in range(640 // 16):
        zbuf[pl.ds(k * 16, 16)] = jnp.zeros((16,), jnp.float32)
    pltpu.sync_copy(zbuf.at[pl.ds(0, DPT)], acc.at[pl.ds(s * DPT, DPT)])
    plsc.subcore_barrier()

    def body(j, carry):
        pltpu.sync_copy(ones_v, acc.at[didx.at[j]], add=True)
        return carry

    lax.fori_loop(0, CHUNKS, body, 0)
    plsc.subcore_barrier()
    pltpu.sync_copy(acc.at[pl.ds(s * DPT, DPT)], zbuf.at[pl.ds(0, DPT)])
    pltpu.sync_copy(
        zbuf.at[pl.ds(0, DPT)],
        out_hbm.at[pl.ds(c * N_ACC + s * DPT, DPT)],
    )


_deg_call = pl.kernel(
    _deg_body,
    out_type=jax.ShapeDtypeStruct((NC * N_ACC,), jnp.float32),
    mesh=_mesh,
    scratch_types=[
        pltpu.VMEM((CHUNKS, CH), jnp.int32),
        pltpu.VMEM((CH,), jnp.float32),
        pltpu.VMEM((640,), jnp.float32),
        pltpu.SemaphoreType.DMA,
        pltpu.VMEM_SHARED((N_ACC,), jnp.float32),
    ],
)


def _mp_body(src_hbm, dst_hbm, y_hbm, out_hbm, sidx, didx, rows0, rows1,
             sem0, sem1, ssem0, ssem1, acc):
    c = lax.axis_index("c")
    s = lax.axis_index("s")
    t = c * NS + s

    def zrow(i, carry):
        for k in range(D // 16):
            rows0[i, pl.ds(k * 16, 16)] = jnp.zeros((16,), jnp.float32)
        return carry

    lax.fori_loop(0, CH, zrow, 0)
    nz = DPT // CH
    tail = DPT - nz * CH
    for k in range(nz):
        pltpu.sync_copy(rows0, acc.at[pl.ds(s * DPT + k * CH, CH)])
    pltpu.sync_copy(
        rows0.at[pl.ds(0, tail)], acc.at[pl.ds(s * DPT + nz * CH, tail)]
    )
    plsc.subcore_barrier()

    pltpu.sync_copy(src_hbm.at[t], sidx)
    pltpu.sync_copy(dst_hbm.at[t], didx)

    def _gather(j, buf, sem):
        pltpu.async_copy(y_hbm.at[sidx.at[pl.ds(j * CH, CH)]], buf, sem)

    def _gwait(j, buf, sem):
        pltpu.make_async_copy(y_hbm.at[sidx.at[pl.ds(j * CH, CH)]], buf, sem).wait()

    def _scat(j, buf, sem):
        pltpu.async_copy(buf, acc.at[didx.at[j]], sem, add=True)

    def _swait(j, buf, sem):
        pltpu.make_async_copy(buf, acc.at[didx.at[j]], sem).wait()

    _gather(0, rows0, sem0)
    _gather(1, rows1, sem1)

    def body(i, carry):
        j0 = 2 * i
        _gwait(j0, rows0, sem0)
        _scat(j0, rows0, ssem0)
        _gwait(j0 + 1, rows1, sem1)
        _scat(j0 + 1, rows1, ssem1)
        _swait(j0, rows0, ssem0)
        _gather(j0 + 2, rows0, sem0)
        _swait(j0 + 1, rows1, ssem1)
        nxt = jnp.where(j0 + 3 < CHUNKS, j0 + 3, CHUNKS - 1)
        _gather(nxt, rows1, sem1)
        return carry

    lax.fori_loop(0, CHUNKS // 2, body, 0)
    _gwait(CHUNKS - 1, rows0, sem0)
    _scat(CHUNKS - 1, rows0, ssem0)
    _gwait(CHUNKS - 1, rows1, sem1)
    _swait(CHUNKS - 1, rows0, ssem0)

    plsc.subcore_barrier()
    for k in range(nz):
        pltpu.sync_copy(acc.at[pl.ds(s * DPT + k * CH, CH)], rows0)
        pltpu.sync_copy(rows0, out_hbm.at[c].at[pl.ds(s * DPT + k * CH, CH)])
    pltpu.sync_copy(
        acc.at[pl.ds(s * DPT + nz * CH, tail)], rows0.at[pl.ds(0, tail)]
    )
    pltpu.sync_copy(
        rows0.at[pl.ds(0, tail)],
        out_hbm.at[c].at[pl.ds(s * DPT + nz * CH, tail)],
    )


_mp_call = pl.kernel(
    _mp_body,
    out_type=jax.ShapeDtypeStruct((NC, N_ACC, D), jnp.float32),
    mesh=_mesh,
    scratch_types=[
        pltpu.VMEM((CHUNKS * CH,), jnp.int32),
        pltpu.VMEM((CHUNKS, CH), jnp.int32),
        pltpu.VMEM((CH, D), jnp.float32),
        pltpu.VMEM((CH, D), jnp.float32),
        pltpu.SemaphoreType.DMA,
        pltpu.SemaphoreType.DMA,
        pltpu.SemaphoreType.DMA,
        pltpu.SemaphoreType.DMA,
        pltpu.VMEM_SHARED((N_ACC, D), jnp.float32),
    ],
)


RB = 2000


def _dis(dp_ref):
    return lax.rsqrt(dp_ref[:, 0:1] + dp_ref[:, 1:2] + 1.0)


def _s1_body(x_ref, w_ref, dp_ref, y_ref):
    y_ref[...] = (
        jnp.dot(x_ref[...], w_ref[...], preferred_element_type=jnp.float32)
        * _dis(dp_ref)
    )


def _s2_body(a_ref, y_ref, dp_ref, b_ref, w_ref, o_ref):
    dis = _dis(dp_ref)
    pre = (a_ref[0] + a_ref[1] + y_ref[...]) * dis + b_ref[...]
    h = jnp.maximum(pre, 0.0)
    o_ref[...] = (
        jnp.dot(h, w_ref[...], preferred_element_type=jnp.float32) * dis
    )


def _s3_body(a_ref, y_ref, dp_ref, b_ref, o_ref):
    dis = _dis(dp_ref)
    z = (a_ref[0] + a_ref[1] + y_ref[...]) * dis + b_ref[...]
    m = jnp.max(z, axis=1, keepdims=True)
    zs = z - m
    lse = jnp.log(jnp.sum(jnp.exp(zs), axis=1, keepdims=True))
    o_ref[...] = zs - lse


def _row_spec(cols):
    return pl.BlockSpec((RB, cols), lambda i: (i, 0))


_full_w = pl.BlockSpec((D, D), lambda i: (0, 0))
_full_b = pl.BlockSpec((1, D), lambda i: (0, 0))
_acc_spec = pl.BlockSpec((NC, RB, D), lambda i: (0, i, 0))

_s1 = pl.pallas_call(
    _s1_body,
    grid=(N // RB,),
    in_specs=[_row_spec(D), _full_w, _row_spec(2)],
    out_specs=_row_spec(D),
    out_shape=jax.ShapeDtypeStruct((N, D), jnp.float32),
)

_s2 = pl.pallas_call(
    _s2_body,
    grid=(N // RB,),
    in_specs=[_acc_spec, _row_spec(D), _row_spec(2), _full_b, _full_w],
    out_specs=_row_spec(D),
    out_shape=jax.ShapeDtypeStruct((N, D), jnp.float32),
)

_s3 = pl.pallas_call(
    _s3_body,
    grid=(N // RB,),
    in_specs=[_acc_spec, _row_spec(D), _row_spec(2), _full_b],
    out_specs=_row_spec(D),
    out_shape=jax.ShapeDtypeStruct((N, D), jnp.float32),
)


def kernel(x, edge_index, W1, b1, W2, b2):
    src = edge_index[0].reshape(NW, CHUNKS * CH)
    dst = edge_index[1].reshape(NW, CHUNKS, CH)

    degp = _deg_call(dst).reshape(NC, N_ACC)
    degp_t = degp[:, :N].T

    y1 = _s1(x, W1, degp_t)
    acc1 = _mp_call(src, dst, y1)
    y2 = _s2(acc1, y1, degp_t, b1.reshape(1, D), W2)
    acc2 = _mp_call(src, dst, y2)
    return _s3(acc2, y2, degp_t, b2.reshape(1, D))


# speedup vs baseline: 1.2577x; 1.2577x over previous
import jax
import jax.numpy as jnp
from jax import lax
from jax.experimental import pallas as pl
from jax.experimental.pallas import tpu as pltpu
from jax.experimental.pallas import tpu_sc as plsc

N = 10000
E = 320000
D = 128

NC = 2
NS = 16
NW = NC * NS

CH = 80
CHUNKS = 125
H = CHUNKS
EPT = CH * CHUNKS
E_PAD = EPT * NW
DPT = 632
N_ACC = DPT * NS

_mesh = plsc.VectorSubcoreMesh(core_axis_name="c", subcore_axis_name="s")


def _deg_body(dst_hbm, out_hbm, didx, ones_v, zbuf, sem, acc):
    c = lax.axis_index("c")
    s = lax.axis_index("s")
    t = c * NS + s
    pltpu.sync_copy(dst_hbm.at[t], didx)
    for k in range(CH // 16):
        ones_v[pl.ds(k * 16, 16)] = jnp.ones((16,), jnp.float32)
    for k in range(640 // 16):
        zbuf[pl.ds(k * 16, 16)] = jnp.zeros((16,), jnp.float32)
    pltpu.sync_copy(zbuf.at[pl.ds(0, DPT)], acc.at[pl.ds(s * DPT, DPT)])
    plsc.subcore_barrier()

    def body(j, carry):
        pltpu.async_copy(ones_v, acc.at[didx.at[j]], sem, add=True)
        return carry

    lax.fori_loop(0, CHUNKS, body, 0)

    def drain(j, carry):
        pltpu.make_async_copy(ones_v, acc.at[didx.at[j]], sem).wait()
        return carry

    lax.fori_loop(0, CHUNKS, drain, 0)
    plsc.subcore_barrier()
    pltpu.sync_copy(acc.at[pl.ds(s * DPT, DPT)], zbuf.at[pl.ds(0, DPT)])
    pltpu.sync_copy(
        zbuf.at[pl.ds(0, DPT)],
        out_hbm.at[pl.ds(c * N_ACC + s * DPT, DPT)],
    )


_deg_call = pl.kernel(
    _deg_body,
    out_type=jax.ShapeDtypeStruct((NC * N_ACC,), jnp.float32),
    mesh=_mesh,
    scratch_types=[
        pltpu.VMEM((CHUNKS, CH), jnp.int32),
        pltpu.VMEM((CH,), jnp.float32),
        pltpu.VMEM((640,), jnp.float32),
        pltpu.SemaphoreType.DMA,
        pltpu.VMEM_SHARED((N_ACC,), jnp.float32),
    ],
)


def _mp_body(src_hbm, dst_hbm, y_hbm, out_hbm, sidx, didx, rows0, rows1,
             sem0, sem1, acc):
    c = lax.axis_index("c")
    s = lax.axis_index("s")
    t = c * NS + s

    def zrow(i, carry):
        for k in range(D // 16):
            rows0[i, pl.ds(k * 16, 16)] = jnp.zeros((16,), jnp.float32)
        return carry

    lax.fori_loop(0, CH, zrow, 0)
    nz = DPT // CH
    tail = DPT - nz * CH
    for k in range(nz):
        pltpu.sync_copy(rows0, acc.at[pl.ds(s * DPT + k * CH, CH)])
    pltpu.sync_copy(
        rows0.at[pl.ds(0, tail)], acc.at[pl.ds(s * DPT + nz * CH, tail)]
    )
    plsc.subcore_barrier()

    pltpu.sync_copy(src_hbm.at[t], sidx)
    pltpu.sync_copy(dst_hbm.at[t], didx)

    def _gather(j, buf, sem):
        pltpu.async_copy(y_hbm.at[sidx.at[pl.ds(j * CH, CH)]], buf, sem)

    def _gwait(j, buf, sem):
        pltpu.make_async_copy(y_hbm.at[sidx.at[pl.ds(j * CH, CH)]], buf, sem).wait()

    _gather(0, rows0, sem0)

    def body(i, carry):
        j0 = 2 * i
        _gather(j0 + 1, rows1, sem1)
        _gwait(j0, rows0, sem0)
        pltpu.sync_copy(rows0, acc.at[didx.at[j0]], add=True)
        _gather(j0 + 2, rows0, sem0)
        _gwait(j0 + 1, rows1, sem1)
        pltpu.sync_copy(rows1, acc.at[didx.at[j0 + 1]], add=True)
        return carry

    lax.fori_loop(0, CHUNKS // 2, body, 0)
    _gwait(CHUNKS - 1, rows0, sem0)
    pltpu.sync_copy(rows0, acc.at[didx.at[CHUNKS - 1]], add=True)

    plsc.subcore_barrier()
    for k in range(nz):
        pltpu.sync_copy(acc.at[pl.ds(s * DPT + k * CH, CH)], rows0)
        pltpu.sync_copy(rows0, out_hbm.at[c].at[pl.ds(s * DPT + k * CH, CH)])
    pltpu.sync_copy(
        acc.at[pl.ds(s * DPT + nz * CH, tail)], rows0.at[pl.ds(0, tail)]
    )
    pltpu.sync_copy(
        rows0.at[pl.ds(0, tail)],
        out_hbm.at[c].at[pl.ds(s * DPT + nz * CH, tail)],
    )


_mp_call = pl.kernel(
    _mp_body,
    out_type=jax.ShapeDtypeStruct((NC, N_ACC, D), jnp.float32),
    mesh=_mesh,
    scratch_types=[
        pltpu.VMEM((CHUNKS * CH,), jnp.int32),
        pltpu.VMEM((CHUNKS, CH), jnp.int32),
        pltpu.VMEM((CH, D), jnp.float32),
        pltpu.VMEM((CH, D), jnp.float32),
        pltpu.SemaphoreType.DMA,
        pltpu.SemaphoreType.DMA,
        pltpu.VMEM_SHARED((N_ACC, D), jnp.float32),
    ],
)


RB = 2000


def _dis(dp_ref):
    return lax.rsqrt(dp_ref[:, 0:1] + dp_ref[:, 1:2] + 1.0)


def _s1_body(x_ref, w_ref, dp_ref, y_ref):
    y_ref[...] = (
        jnp.dot(x_ref[...], w_ref[...], preferred_element_type=jnp.float32)
        * _dis(dp_ref)
    )


def _s2_body(a_ref, y_ref, dp_ref, b_ref, w_ref, o_ref):
    dis = _dis(dp_ref)
    pre = (a_ref[0] + a_ref[1] + y_ref[...]) * dis + b_ref[...]
    h = jnp.maximum(pre, 0.0)
    o_ref[...] = (
        jnp.dot(h, w_ref[...], preferred_element_type=jnp.float32) * dis
    )


def _s3_body(a_ref, y_ref, dp_ref, b_ref, o_ref):
    dis = _dis(dp_ref)
    z = (a_ref[0] + a_ref[1] + y_ref[...]) * dis + b_ref[...]
    m = jnp.max(z, axis=1, keepdims=True)
    zs = z - m
    lse = jnp.log(jnp.sum(jnp.exp(zs), axis=1, keepdims=True))
    o_ref[...] = zs - lse


def _row_spec(cols):
    return pl.BlockSpec((RB, cols), lambda i: (i, 0))


_full_w = pl.BlockSpec((D, D), lambda i: (0, 0))
_full_b = pl.BlockSpec((1, D), lambda i: (0, 0))
_acc_spec = pl.BlockSpec((NC, RB, D), lambda i: (0, i, 0))

_s1 = pl.pallas_call(
    _s1_body,
    grid=(N // RB,),
    in_specs=[_row_spec(D), _full_w, _row_spec(2)],
    out_specs=_row_spec(D),
    out_shape=jax.ShapeDtypeStruct((N, D), jnp.float32),
)

_s2 = pl.pallas_call(
    _s2_body,
    grid=(N // RB,),
    in_specs=[_acc_spec, _row_spec(D), _row_spec(2), _full_b, _full_w],
    out_specs=_row_spec(D),
    out_shape=jax.ShapeDtypeStruct((N, D), jnp.float32),
)

_s3 = pl.pallas_call(
    _s3_body,
    grid=(N // RB,),
    in_specs=[_acc_spec, _row_spec(D), _row_spec(2), _full_b],
    out_specs=_row_spec(D),
    out_shape=jax.ShapeDtypeStruct((N, D), jnp.float32),
)


def kernel(x, edge_index, W1, b1, W2, b2):
    src = edge_index[0].reshape(NW, CHUNKS * CH)
    dst = edge_index[1].reshape(NW, CHUNKS, CH)

    degp = _deg_call(dst).reshape(NC, N_ACC)
    degp_t = degp[:, :N].T

    y1 = _s1(x, W1, degp_t)
    acc1 = _mp_call(src, dst, y1)
    y2 = _s2(acc1, y1, degp_t, b1.reshape(1, D), W2)
    acc2 = _mp_call(src, dst, y2)
    return _s3(acc2, y2, degp_t, b2.reshape(1, D))


# speedup vs baseline: 1.3003x; 1.0339x over previous
import jax
import jax.numpy as jnp
from jax import lax
from jax.experimental import pallas as pl
from jax.experimental.pallas import tpu as pltpu
from jax.experimental.pallas import tpu_sc as plsc

N = 10000
E = 320000
D = 128

NC = 2
NS = 16
NW = NC * NS

CH = 80
CHUNKS = 125
H = CHUNKS
EPT = CH * CHUNKS
E_PAD = EPT * NW
DPT = 632
N_ACC = DPT * NS

_mesh = plsc.VectorSubcoreMesh(core_axis_name="c", subcore_axis_name="s")


def _deg_body(dst_hbm, out_hbm, didx, ones_v, zbuf, sem, acc):
    c = lax.axis_index("c")
    s = lax.axis_index("s")
    t = c * NS + s
    pltpu.sync_copy(dst_hbm.at[t], didx)
    for k in range(CH // 16):
        ones_v[pl.ds(k * 16, 16)] = jnp.ones((16,), jnp.float32)
    for k in range(640 // 16):
        zbuf[pl.ds(k * 16, 16)] = jnp.zeros((16,), jnp.float32)
    pltpu.sync_copy(zbuf.at[pl.ds(0, DPT)], acc.at[pl.ds(s * DPT, DPT)])
    plsc.subcore_barrier()

    def body(j, carry):
        pltpu.async_copy(ones_v, acc.at[didx.at[j]], sem, add=True)
        return carry

    lax.fori_loop(0, CHUNKS, body, 0)

    def drain(j, carry):
        pltpu.make_async_copy(ones_v, acc.at[didx.at[j]], sem).wait()
        return carry

    lax.fori_loop(0, CHUNKS, drain, 0)
    plsc.subcore_barrier()
    pltpu.sync_copy(acc.at[pl.ds(s * DPT, DPT)], zbuf.at[pl.ds(0, DPT)])
    pltpu.sync_copy(
        zbuf.at[pl.ds(0, DPT)],
        out_hbm.at[pl.ds(c * N_ACC + s * DPT, DPT)],
    )


_deg_call = pl.kernel(
    _deg_body,
    out_type=jax.ShapeDtypeStruct((NC * N_ACC,), jnp.float32),
    mesh=_mesh,
    scratch_types=[
        pltpu.VMEM((CHUNKS, CH), jnp.int32),
        pltpu.VMEM((CH,), jnp.float32),
        pltpu.VMEM((640,), jnp.float32),
        pltpu.SemaphoreType.DMA,
        pltpu.VMEM_SHARED((N_ACC,), jnp.float32),
    ],
)


def _mp_body(src_hbm, dst_hbm, y_hbm, out_hbm, sidx, didx, rows0, rows1,
             sem0, sem1, acc):
    c = lax.axis_index("c")
    s = lax.axis_index("s")
    t = c * NS + s

    def zrow(i, carry):
        for k in range(D // 16):
            rows0[i, pl.ds(k * 16, 16)] = jnp.zeros((16,), jnp.float32)
        return carry

    lax.fori_loop(0, CH, zrow, 0)
    nz = DPT // CH
    tail = DPT - nz * CH
    for k in range(nz):
        pltpu.async_copy(rows0, acc.at[pl.ds(s * DPT + k * CH, CH)], sem0)
    pltpu.async_copy(
        rows0.at[pl.ds(0, tail)], acc.at[pl.ds(s * DPT + nz * CH, tail)], sem0
    )
    pltpu.sync_copy(src_hbm.at[t], sidx)
    pltpu.sync_copy(dst_hbm.at[t], didx)
    for k in range(nz):
        pltpu.make_async_copy(
            rows0, acc.at[pl.ds(s * DPT + k * CH, CH)], sem0
        ).wait()
    pltpu.make_async_copy(
        rows0.at[pl.ds(0, tail)], acc.at[pl.ds(s * DPT + nz * CH, tail)], sem0
    ).wait()
    plsc.subcore_barrier()

    def _gather(j, buf, sem):
        pltpu.async_copy(y_hbm.at[sidx.at[pl.ds(j * CH, CH)]], buf, sem)

    def _gwait(j, buf, sem):
        pltpu.make_async_copy(y_hbm.at[sidx.at[pl.ds(j * CH, CH)]], buf, sem).wait()

    _gather(0, rows0, sem0)

    def body(i, carry):
        j0 = 2 * i
        _gather(j0 + 1, rows1, sem1)
        _gwait(j0, rows0, sem0)
        pltpu.sync_copy(rows0, acc.at[didx.at[j0]], add=True)
        _gather(j0 + 2, rows0, sem0)
        _gwait(j0 + 1, rows1, sem1)
        pltpu.sync_copy(rows1, acc.at[didx.at[j0 + 1]], add=True)
        return carry

    lax.fori_loop(0, CHUNKS // 2, body, 0)
    _gwait(CHUNKS - 1, rows0, sem0)
    pltpu.sync_copy(rows0, acc.at[didx.at[CHUNKS - 1]], add=True)

    plsc.subcore_barrier()
    blocks = [(k * CH, CH) for k in range(nz)] + [(nz * CH, tail)]
    for k, (off, ln) in enumerate(blocks):
        buf, sem = (rows0, sem0) if k % 2 == 0 else (rows1, sem1)
        if k >= 2:
            poff, pln = blocks[k - 2]
            pltpu.make_async_copy(
                buf.at[pl.ds(0, pln)],
                out_hbm.at[c].at[pl.ds(s * DPT + poff, pln)],
                sem,
            ).wait()
        pltpu.sync_copy(acc.at[pl.ds(s * DPT + off, ln)], buf.at[pl.ds(0, ln)])
        pltpu.async_copy(
            buf.at[pl.ds(0, ln)],
            out_hbm.at[c].at[pl.ds(s * DPT + off, ln)],
            sem,
        )
    for k in (len(blocks) - 2, len(blocks) - 1):
        off, ln = blocks[k]
        buf, sem = (rows0, sem0) if k % 2 == 0 else (rows1, sem1)
        pltpu.make_async_copy(
            buf.at[pl.ds(0, ln)],
            out_hbm.at[c].at[pl.ds(s * DPT + off, ln)],
            sem,
        ).wait()


_mp_call = pl.kernel(
    _mp_body,
    out_type=jax.ShapeDtypeStruct((NC, N_ACC, D), jnp.float32),
    mesh=_mesh,
    scratch_types=[
        pltpu.VMEM((CHUNKS * CH,), jnp.int32),
        pltpu.VMEM((CHUNKS, CH), jnp.int32),
        pltpu.VMEM((CH, D), jnp.float32),
        pltpu.VMEM((CH, D), jnp.float32),
        pltpu.SemaphoreType.DMA,
        pltpu.SemaphoreType.DMA,
        pltpu.VMEM_SHARED((N_ACC, D), jnp.float32),
    ],
)


RB = 2000


def _dis(dp_ref):
    return lax.rsqrt(dp_ref[:, 0:1] + dp_ref[:, 1:2] + 1.0)


def _s1_body(x_ref, w_ref, dp_ref, y_ref):
    y_ref[...] = (
        jnp.dot(x_ref[...], w_ref[...], preferred_element_type=jnp.float32)
        * _dis(dp_ref)
    )


def _s2_body(a_ref, y_ref, dp_ref, b_ref, w_ref, o_ref):
    dis = _dis(dp_ref)
    pre = (a_ref[0] + a_ref[1] + y_ref[...]) * dis + b_ref[...]
    h = jnp.maximum(pre, 0.0)
    o_ref[...] = (
        jnp.dot(h, w_ref[...], preferred_element_type=jnp.float32) * dis
    )


def _s3_body(a_ref, y_ref, dp_ref, b_ref, o_ref):
    dis = _dis(dp_ref)
    z = (a_ref[0] + a_ref[1] + y_ref[...]) * dis + b_ref[...]
    m = jnp.max(z, axis=1, keepdims=True)
    zs = z - m
    lse = jnp.log(jnp.sum(jnp.exp(zs), axis=1, keepdims=True))
    o_ref[...] = zs - lse


def _row_spec(cols):
    return pl.BlockSpec((RB, cols), lambda i: (i, 0))


_full_w = pl.BlockSpec((D, D), lambda i: (0, 0))
_full_b = pl.BlockSpec((1, D), lambda i: (0, 0))
_acc_spec = pl.BlockSpec((NC, RB, D), lambda i: (0, i, 0))

_s1 = pl.pallas_call(
    _s1_body,
    grid=(N // RB,),
    in_specs=[_row_spec(D), _full_w, _row_spec(2)],
    out_specs=_row_spec(D),
    out_shape=jax.ShapeDtypeStruct((N, D), jnp.float32),
)

_s2 = pl.pallas_call(
    _s2_body,
    grid=(N // RB,),
    in_specs=[_acc_spec, _row_spec(D), _row_spec(2), _full_b, _full_w],
    out_specs=_row_spec(D),
    out_shape=jax.ShapeDtypeStruct((N, D), jnp.float32),
)

_s3 = pl.pallas_call(
    _s3_body,
    grid=(N // RB,),
    in_specs=[_acc_spec, _row_spec(D), _row_spec(2), _full_b],
    out_specs=_row_spec(D),
    out_shape=jax.ShapeDtypeStruct((N, D), jnp.float32),
)


def kernel(x, edge_index, W1, b1, W2, b2):
    src = edge_index[0].reshape(NW, CHUNKS * CH)
    dst = edge_index[1].reshape(NW, CHUNKS, CH)

    degp = _deg_call(dst).reshape(NC, N_ACC)
    degp_t = degp[:, :N].T

    y1 = _s1(x, W1, degp_t)
    acc1 = _mp_call(src, dst, y1)
    y2 = _s2(acc1, y1, degp_t, b1.reshape(1, D), W2)
    acc2 = _mp_call(src, dst, y2)
    return _s3(acc2, y2, degp_t, b2.reshape(1, D))
